# baseline (speedup 1.0000x reference)
import functools

import jax
import jax.numpy as jnp
from jax import lax
from jax.experimental import pallas as pl
from jax.experimental.pallas import tpu as pltpu
from jax.experimental.pallas import tpu_sc as plsc

IDIM = 1024
HID = 4096
NE = 8
T = 8192
BLK = 256
NROWS = 18432
NBLK = NROWS // BLK

NW = 32
CHUNK = T // NW
DSUB = 32
CSUB = 16

def _sc_mesh():
    return plsc.VectorSubcoreMesh(core_axis_name="c", subcore_axis_name="s")


def _route_meta_body(x_ref, gw_ref, d0_ref, d1_ref, w0_ref, w1_ref, ps_ref):
    logits = lax.dot_general(gw_ref[...], x_ref[...], (((1,), (1,)), ((), ())),
                             preferred_element_type=jnp.float32)
    row = lax.broadcasted_iota(jnp.int32, (NE, T), 0)
    m1 = jnp.max(logits, axis=0, keepdims=True)
    e0 = jnp.min(jnp.where(logits == m1, row, NE), axis=0, keepdims=True)
    mask0 = row == e0
    l2 = jnp.where(mask0, -jnp.inf, logits)
    m2 = jnp.max(l2, axis=0, keepdims=True)
    e1 = jnp.min(jnp.where(l2 == m2, row, NE), axis=0, keepdims=True)
    mask1 = row == e1

    w0 = 1.0 / (1.0 + jnp.exp(m2 - m1))
    w0_ref[...] = w0[0]
    w1_ref[...] = (1.0 - w0)[0]

    s = mask0.astype(jnp.int32) + mask1.astype(jnp.int32)
    c = s
    k = 1
    while k < T:
        c = c + jnp.concatenate([jnp.zeros((NE, k), jnp.int32), c[:, :-k]], axis=1)
        k *= 2
    excl = c - s

    tot = lax.slice(c, (0, T - 1), (NE, T))
    pc = ((tot + (BLK - 1)) // BLK) * BLK
    ps = pc
    k = 1
    while k < NE:
        ps = ps + jnp.concatenate([jnp.zeros((k, 1), jnp.int32), ps[:-k]], axis=0)
        k *= 2
    ps = ps - pc
    ps_ref[...] = ps

    d0_ref[...] = jnp.sum(jnp.where(mask0, excl + ps, 0), axis=0)
    d1_ref[...] = jnp.sum(jnp.where(mask1, excl + ps, 0), axis=0)


def _route_meta(x, gate_w):
    return pl.pallas_call(
        _route_meta_body,
        out_shape=(
            jax.ShapeDtypeStruct((T,), jnp.int32),
            jax.ShapeDtypeStruct((T,), jnp.int32),
            jax.ShapeDtypeStruct((T,), jnp.float32),
            jax.ShapeDtypeStruct((T,), jnp.float32),
            jax.ShapeDtypeStruct((NE, 1), jnp.int32),
        ),
        compiler_params=pltpu.CompilerParams(
            vmem_limit_bytes=100 * 1024 * 1024,
        ),
    )(x, gate_w)


def _dispatch_body(x_hbm, d0_hbm, d1_hbm, xs_hbm, xv, i0, i1, sem):
    wid = lax.axis_index("s") * 2 + lax.axis_index("c")
    base = wid * CHUNK

    def step(j, _):
        off = base + j * DSUB
        pltpu.sync_copy(d0_hbm.at[pl.ds(off, DSUB)], i0)
        pltpu.sync_copy(d1_hbm.at[pl.ds(off, DSUB)], i1)
        pltpu.sync_copy(x_hbm.at[pl.ds(off, DSUB)], xv)
        pltpu.async_copy(xv, xs_hbm.at[i0], sem).wait()
        pltpu.async_copy(xv, xs_hbm.at[i1], sem).wait()
        return 0

    lax.fori_loop(0, CHUNK // DSUB, step, 0)


def _dispatch(x, d0, d1):
    f = pl.kernel(
        _dispatch_body,
        out_type=jax.ShapeDtypeStruct((NROWS, IDIM), jnp.float32),
        mesh=_sc_mesh(),
        scratch_types=[
            pltpu.VMEM((DSUB, IDIM), jnp.float32),
            pltpu.VMEM((DSUB,), jnp.int32),
            pltpu.VMEM((DSUB,), jnp.int32),
            pltpu.SemaphoreType.DMA,
        ],
    )
    return f(x, d0, d1)


def _gmm_body(be_ref, xs_ref, w1_ref, w2_ref, out_ref):
    xb = xs_ref[...].astype(jnp.bfloat16)
    h = lax.dot_general(xb, w1_ref[0], (((1,), (0,)), ((), ())),
                        preferred_element_type=jnp.float32)
    h = jnp.maximum(h.astype(jnp.bfloat16), 0)
    out_ref[...] = lax.dot_general(h, w2_ref[0], (((1,), (0,)), ((), ())),
                                   preferred_element_type=jnp.float32)


def _gmm(be, xsorted, w1t, w2t):
    return pl.pallas_call(
        _gmm_body,
        grid_spec=pltpu.PrefetchScalarGridSpec(
            num_scalar_prefetch=1,
            grid=(NBLK,),
            in_specs=[
                pl.BlockSpec((BLK, IDIM), lambda b, be_ref: (b, 0)),
                pl.BlockSpec((1, IDIM, HID), lambda b, be_ref: (be_ref[b], 0, 0)),
                pl.BlockSpec((1, HID, IDIM), lambda b, be_ref: (be_ref[b], 0, 0)),
            ],
            out_specs=pl.BlockSpec((BLK, IDIM), lambda b, be_ref: (b, 0)),
        ),
        out_shape=jax.ShapeDtypeStruct((NROWS, IDIM), jnp.float32),
        compiler_params=pltpu.CompilerParams(
            dimension_semantics=("arbitrary",),
            vmem_limit_bytes=100 * 1024 * 1024,
        ),
    )(be, xsorted, w1t, w2t)


def _combine_body(ys_hbm, d0_hbm, d1_hbm, w0_hbm, w1_hbm, out_hbm,
                  y0v, y1v, ov, i0, i1, wv0, wv1, sem):
    wid = lax.axis_index("s") * 2 + lax.axis_index("c")
    base = wid * CHUNK
    lane = lax.iota(jnp.int32, 16)

    def step(j, _):
        off = base + j * CSUB
        pltpu.sync_copy(d0_hbm.at[pl.ds(off, CSUB)], i0)
        pltpu.sync_copy(d1_hbm.at[pl.ds(off, CSUB)], i1)
        pltpu.sync_copy(w0_hbm.at[pl.ds(off, CSUB)], wv0)
        pltpu.sync_copy(w1_hbm.at[pl.ds(off, CSUB)], wv1)
        pltpu.async_copy(ys_hbm.at[i0], y0v, sem).wait()
        pltpu.async_copy(ys_hbm.at[i1], y1v, sem).wait()
        wa = wv0[...]
        wb = wv1[...]

        def tok(t, _):
            a = jnp.broadcast_to(jnp.sum(jnp.where(lane == t, wa, 0.0)), (16,))
            b = jnp.broadcast_to(jnp.sum(jnp.where(lane == t, wb, 0.0)), (16,))

            def colgrp(g, _):
                for u in range(8):
                    sl = pl.ds(g * 128 + u * 16, 16)
                    ov[t, sl] = a * y0v[t, sl] + b * y1v[t, sl]
                return 0

            lax.fori_loop(0, IDIM // 128, colgrp, 0)
            return 0

        lax.fori_loop(0, CSUB, tok, 0)
        pltpu.sync_copy(ov, out_hbm.at[pl.ds(off, CSUB)])
        return 0

    lax.fori_loop(0, CHUNK // CSUB, step, 0)


def _combine(ysorted, d0, d1, wt0, wt1):
    f = pl.kernel(
        _combine_body,
        out_type=jax.ShapeDtypeStruct((T, IDIM), jnp.float32),
        mesh=_sc_mesh(),
        scratch_types=[
            pltpu.VMEM((CSUB, IDIM), jnp.float32),
            pltpu.VMEM((CSUB, IDIM), jnp.float32),
            pltpu.VMEM((CSUB, IDIM), jnp.float32),
            pltpu.VMEM((CSUB,), jnp.int32),
            pltpu.VMEM((CSUB,), jnp.int32),
            pltpu.VMEM((CSUB,), jnp.float32),
            pltpu.VMEM((CSUB,), jnp.float32),
            pltpu.SemaphoreType.DMA,
        ],
        compiler_params=pltpu.CompilerParams(needs_layout_passes=False),
    )
    return f(ysorted, d0, d1, wt0, wt1)


def kernel(xs, gate_w, w1, w2):
    x = xs.reshape(-1, IDIM)
    d0, d1, wt0, wt1, ps2 = _route_meta(x, gate_w)
    ps = ps2[:, 0]
    starts = jnp.arange(NBLK, dtype=jnp.int32) * BLK
    be = jnp.sum((starts[:, None] >= ps[None, :]).astype(jnp.int32), axis=1) - 1
    w1t = jnp.swapaxes(w1, 1, 2).astype(jnp.bfloat16)
    w2t = jnp.swapaxes(w2, 1, 2).astype(jnp.bfloat16)
    xsorted = _dispatch(x, d0, d1)
    ysorted = _gmm(be, xsorted, w1t, w2t)
    out = _combine(ysorted, d0, d1, wt0, wt1)
    return out.reshape(xs.shape)


# speedup vs baseline: 1.0762x; 1.0762x over previous
import functools

import jax
import jax.numpy as jnp
from jax import lax
from jax.experimental import pallas as pl
from jax.experimental.pallas import tpu as pltpu
from jax.experimental.pallas import tpu_sc as plsc

IDIM = 1024
HID = 4096
NE = 8
T = 8192
BLK = 256
NROWS = 18432
NBLK = NROWS // BLK

NW = 32
CHUNK = T // NW
DSUB = 32
CSUB = 16

def _sc_mesh():
    return plsc.VectorSubcoreMesh(core_axis_name="c", subcore_axis_name="s")


def _route_meta_body(x_ref, gw_ref, d0_ref, d1_ref, w0_ref, w1_ref, ps_ref):
    logits = lax.dot_general(gw_ref[...], x_ref[...], (((1,), (1,)), ((), ())),
                             preferred_element_type=jnp.float32)
    row = lax.broadcasted_iota(jnp.int32, (NE, T), 0)
    m1 = jnp.max(logits, axis=0, keepdims=True)
    e0 = jnp.min(jnp.where(logits == m1, row, NE), axis=0, keepdims=True)
    mask0 = row == e0
    l2 = jnp.where(mask0, -jnp.inf, logits)
    m2 = jnp.max(l2, axis=0, keepdims=True)
    e1 = jnp.min(jnp.where(l2 == m2, row, NE), axis=0, keepdims=True)
    mask1 = row == e1

    w0 = 1.0 / (1.0 + jnp.exp(m2 - m1))
    w0_ref[...] = w0[0]
    w1_ref[...] = (1.0 - w0)[0]

    s = mask0.astype(jnp.int32) + mask1.astype(jnp.int32)
    c = s
    k = 1
    while k < T:
        c = c + jnp.concatenate([jnp.zeros((NE, k), jnp.int32), c[:, :-k]], axis=1)
        k *= 2
    excl = c - s

    tot = lax.slice(c, (0, T - 1), (NE, T))
    pc = ((tot + (BLK - 1)) // BLK) * BLK
    ps = pc
    k = 1
    while k < NE:
        ps = ps + jnp.concatenate([jnp.zeros((k, 1), jnp.int32), ps[:-k]], axis=0)
        k *= 2
    ps = ps - pc
    ps_ref[...] = ps

    d0_ref[...] = jnp.sum(jnp.where(mask0, excl + ps, 0), axis=0)
    d1_ref[...] = jnp.sum(jnp.where(mask1, excl + ps, 0), axis=0)


def _route_meta(x, gate_w):
    return pl.pallas_call(
        _route_meta_body,
        out_shape=(
            jax.ShapeDtypeStruct((T,), jnp.int32),
            jax.ShapeDtypeStruct((T,), jnp.int32),
            jax.ShapeDtypeStruct((T,), jnp.float32),
            jax.ShapeDtypeStruct((T,), jnp.float32),
            jax.ShapeDtypeStruct((NE, 1), jnp.int32),
        ),
        compiler_params=pltpu.CompilerParams(
            vmem_limit_bytes=100 * 1024 * 1024,
        ),
    )(x, gate_w)


def _dispatch_body(x_hbm, d0_hbm, d1_hbm, xs_hbm, xv, i0, i1, sem):
    wid = lax.axis_index("s") * 2 + lax.axis_index("c")
    base = wid * CHUNK

    def step(j, _):
        off = base + j * DSUB
        pltpu.sync_copy(d0_hbm.at[pl.ds(off, DSUB)], i0)
        pltpu.sync_copy(d1_hbm.at[pl.ds(off, DSUB)], i1)
        pltpu.sync_copy(x_hbm.at[pl.ds(off, DSUB)], xv)
        pltpu.async_copy(xv, xs_hbm.at[i0], sem).wait()
        pltpu.async_copy(xv, xs_hbm.at[i1], sem).wait()
        return 0

    lax.fori_loop(0, CHUNK // DSUB, step, 0)


def _dispatch(x, d0, d1):
    f = pl.kernel(
        _dispatch_body,
        out_type=jax.ShapeDtypeStruct((NROWS, IDIM), jnp.float32),
        mesh=_sc_mesh(),
        scratch_types=[
            pltpu.VMEM((DSUB, IDIM), jnp.float32),
            pltpu.VMEM((DSUB,), jnp.int32),
            pltpu.VMEM((DSUB,), jnp.int32),
            pltpu.SemaphoreType.DMA,
        ],
    )
    return f(x, d0, d1)


def _gmm_body(be_ref, xs_ref, w1_ref, w2_ref, out_ref):
    xb = xs_ref[...].astype(jnp.bfloat16)
    h = lax.dot_general(xb, w1_ref[0], (((1,), (1,)), ((), ())),
                        preferred_element_type=jnp.float32)
    h = jnp.maximum(h.astype(jnp.bfloat16), 0)
    out_ref[...] = lax.dot_general(h, w2_ref[0], (((1,), (1,)), ((), ())),
                                   preferred_element_type=jnp.float32)


def _gmm(be, xsorted, w1t, w2t):
    return pl.pallas_call(
        _gmm_body,
        grid_spec=pltpu.PrefetchScalarGridSpec(
            num_scalar_prefetch=1,
            grid=(NBLK,),
            in_specs=[
                pl.BlockSpec((BLK, IDIM), lambda b, be_ref: (b, 0)),
                pl.BlockSpec((1, HID, IDIM), lambda b, be_ref: (be_ref[b], 0, 0)),
                pl.BlockSpec((1, IDIM, HID), lambda b, be_ref: (be_ref[b], 0, 0)),
            ],
            out_specs=pl.BlockSpec((BLK, IDIM), lambda b, be_ref: (b, 0)),
        ),
        out_shape=jax.ShapeDtypeStruct((NROWS, IDIM), jnp.float32),
        compiler_params=pltpu.CompilerParams(
            dimension_semantics=("arbitrary",),
            vmem_limit_bytes=100 * 1024 * 1024,
        ),
    )(be, xsorted, w1t, w2t)


def _combine_body(ys_hbm, d0_hbm, d1_hbm, w0_hbm, w1_hbm, out_hbm,
                  y0v, y1v, ov, i0, i1, wv0, wv1, sem):
    wid = lax.axis_index("s") * 2 + lax.axis_index("c")
    base = wid * CHUNK
    lane = lax.iota(jnp.int32, 16)

    def step(j, _):
        off = base + j * CSUB
        pltpu.sync_copy(d0_hbm.at[pl.ds(off, CSUB)], i0)
        pltpu.sync_copy(d1_hbm.at[pl.ds(off, CSUB)], i1)
        pltpu.sync_copy(w0_hbm.at[pl.ds(off, CSUB)], wv0)
        pltpu.sync_copy(w1_hbm.at[pl.ds(off, CSUB)], wv1)
        pltpu.async_copy(ys_hbm.at[i0], y0v, sem).wait()
        pltpu.async_copy(ys_hbm.at[i1], y1v, sem).wait()
        wa = wv0[...]
        wb = wv1[...]

        def tok(t, _):
            a = jnp.broadcast_to(jnp.sum(jnp.where(lane == t, wa, 0.0)), (16,))
            b = jnp.broadcast_to(jnp.sum(jnp.where(lane == t, wb, 0.0)), (16,))

            def colgrp(g, _):
                for u in range(8):
                    sl = pl.ds(g * 128 + u * 16, 16)
                    ov[t, sl] = a * y0v[t, sl] + b * y1v[t, sl]
                return 0

            lax.fori_loop(0, IDIM // 128, colgrp, 0)
            return 0

        lax.fori_loop(0, CSUB, tok, 0)
        pltpu.sync_copy(ov, out_hbm.at[pl.ds(off, CSUB)])
        return 0

    lax.fori_loop(0, CHUNK // CSUB, step, 0)


def _combine(ysorted, d0, d1, wt0, wt1):
    f = pl.kernel(
        _combine_body,
        out_type=jax.ShapeDtypeStruct((T, IDIM), jnp.float32),
        mesh=_sc_mesh(),
        scratch_types=[
            pltpu.VMEM((CSUB, IDIM), jnp.float32),
            pltpu.VMEM((CSUB, IDIM), jnp.float32),
            pltpu.VMEM((CSUB, IDIM), jnp.float32),
            pltpu.VMEM((CSUB,), jnp.int32),
            pltpu.VMEM((CSUB,), jnp.int32),
            pltpu.VMEM((CSUB,), jnp.float32),
            pltpu.VMEM((CSUB,), jnp.float32),
            pltpu.SemaphoreType.DMA,
        ],
        compiler_params=pltpu.CompilerParams(needs_layout_passes=False),
    )
    return f(ysorted, d0, d1, wt0, wt1)


def kernel(xs, gate_w, w1, w2):
    x = xs.reshape(-1, IDIM)
    d0, d1, wt0, wt1, ps2 = _route_meta(x, gate_w)
    ps = ps2[:, 0]
    starts = jnp.arange(NBLK, dtype=jnp.int32) * BLK
    be = jnp.sum((starts[:, None] >= ps[None, :]).astype(jnp.int32), axis=1) - 1
    xsorted = _dispatch(x, d0, d1)
    ysorted = _gmm(be, xsorted, w1.astype(jnp.bfloat16), w2.astype(jnp.bfloat16))
    out = _combine(ysorted, d0, d1, wt0, wt1)
    return out.reshape(xs.shape)


# speedup vs baseline: 1.1756x; 1.0924x over previous
import functools

import jax
import jax.numpy as jnp
from jax import lax
from jax.experimental import pallas as pl
from jax.experimental.pallas import tpu as pltpu
from jax.experimental.pallas import tpu_sc as plsc

IDIM = 1024
HID = 4096
NE = 8
T = 8192
BLK = 256
NROWS = 18432
NBLK = NROWS // BLK

NW = 32
CHUNK = T // NW
SUB = 16
NSTEP = CHUNK // SUB

def _sc_mesh():
    return plsc.VectorSubcoreMesh(core_axis_name="c", subcore_axis_name="s")


def _route_meta_body(x_ref, gw_ref, d0_ref, d1_ref, w0_ref, w1_ref, ps_ref):
    logits = lax.dot_general(gw_ref[...], x_ref[...], (((1,), (1,)), ((), ())),
                             preferred_element_type=jnp.float32)
    row = lax.broadcasted_iota(jnp.int32, (NE, T), 0)
    m1 = jnp.max(logits, axis=0, keepdims=True)
    e0 = jnp.min(jnp.where(logits == m1, row, NE), axis=0, keepdims=True)
    mask0 = row == e0
    l2 = jnp.where(mask0, -jnp.inf, logits)
    m2 = jnp.max(l2, axis=0, keepdims=True)
    e1 = jnp.min(jnp.where(l2 == m2, row, NE), axis=0, keepdims=True)
    mask1 = row == e1

    w0 = 1.0 / (1.0 + jnp.exp(m2 - m1))
    w0_ref[...] = w0[0]
    w1_ref[...] = (1.0 - w0)[0]

    s = mask0.astype(jnp.int32) + mask1.astype(jnp.int32)
    c = s
    k = 1
    while k < T:
        c = c + jnp.concatenate([jnp.zeros((NE, k), jnp.int32), c[:, :-k]], axis=1)
        k *= 2
    excl = c - s

    tot = lax.slice(c, (0, T - 1), (NE, T))
    pc = ((tot + (BLK - 1)) // BLK) * BLK
    ps = pc
    k = 1
    while k < NE:
        ps = ps + jnp.concatenate([jnp.zeros((k, 1), jnp.int32), ps[:-k]], axis=0)
        k *= 2
    ps = ps - pc
    ps_ref[...] = ps

    d0_ref[...] = jnp.sum(jnp.where(mask0, excl + ps, 0), axis=0)
    d1_ref[...] = jnp.sum(jnp.where(mask1, excl + ps, 0), axis=0)


def _route_meta(x, gate_w):
    return pl.pallas_call(
        _route_meta_body,
        out_shape=(
            jax.ShapeDtypeStruct((T,), jnp.int32),
            jax.ShapeDtypeStruct((T,), jnp.int32),
            jax.ShapeDtypeStruct((T,), jnp.float32),
            jax.ShapeDtypeStruct((T,), jnp.float32),
            jax.ShapeDtypeStruct((NE, 1), jnp.int32),
        ),
        compiler_params=pltpu.CompilerParams(
            vmem_limit_bytes=100 * 1024 * 1024,
        ),
    )(x, gate_w)


def _dispatch_body(x_hbm, d0_hbm, d1_hbm, xs_hbm,
                   i0all, i1all, xvA, xvB, sgA, sgB, scA, scB):
    wid = lax.axis_index("s") * 2 + lax.axis_index("c")
    base = wid * CHUNK
    pltpu.sync_copy(d0_hbm.at[pl.ds(base, CHUNK)], i0all)
    pltpu.sync_copy(d1_hbm.at[pl.ds(base, CHUNK)], i1all)

    def stage(xv, sg, s):
        pltpu.async_copy(x_hbm.at[pl.ds(base + s * SUB, SUB)], xv, sg)

    def scatter(xv, sg, sc, s):
        pltpu.make_async_copy(x_hbm.at[pl.ds(base, SUB)], xv, sg).wait()
        iv0 = i0all[pl.ds(s * SUB, SUB)]
        iv1 = i1all[pl.ds(s * SUB, SUB)]
        pltpu.async_copy(xv, xs_hbm.at[iv0], sc)
        pltpu.async_copy(xv, xs_hbm.at[iv1], sc)

    def drain(xv, sc):
        pltpu.make_async_copy(xv, xs_hbm.at[i0all[pl.ds(0, SUB)]], sc).wait()
        pltpu.make_async_copy(xv, xs_hbm.at[i0all[pl.ds(0, SUB)]], sc).wait()

    stage(xvA, sgA, 0)
    stage(xvB, sgB, 1)

    def step(j2, _):
        sA = 2 * j2
        scatter(xvA, sgA, scA, sA)

        @pl.when(j2 < NSTEP // 2 - 1)
        def _():
            drain(xvA, scA)
            stage(xvA, sgA, sA + 2)

        scatter(xvB, sgB, scB, sA + 1)

        @pl.when(j2 < NSTEP // 2 - 1)
        def _():
            drain(xvB, scB)
            stage(xvB, sgB, sA + 3)

        return 0

    lax.fori_loop(0, NSTEP // 2, step, 0)
    drain(xvA, scA)
    drain(xvB, scB)


def _dispatch(x, d0, d1):
    f = pl.kernel(
        _dispatch_body,
        out_type=jax.ShapeDtypeStruct((NROWS, IDIM), jnp.float32),
        mesh=_sc_mesh(),
        scratch_types=[
            pltpu.VMEM((CHUNK,), jnp.int32),
            pltpu.VMEM((CHUNK,), jnp.int32),
            pltpu.VMEM((SUB, IDIM), jnp.float32),
            pltpu.VMEM((SUB, IDIM), jnp.float32),
            pltpu.SemaphoreType.DMA,
            pltpu.SemaphoreType.DMA,
            pltpu.SemaphoreType.DMA,
            pltpu.SemaphoreType.DMA,
        ],
    )
    return f(x, d0, d1)


def _gmm_body(be_ref, xs_ref, w1_ref, w2_ref, out_ref):
    xb = xs_ref[...].astype(jnp.bfloat16)
    h = lax.dot_general(xb, w1_ref[0], (((1,), (1,)), ((), ())),
                        preferred_element_type=jnp.float32)
    h = jnp.maximum(h.astype(jnp.bfloat16), 0)
    out_ref[...] = lax.dot_general(h, w2_ref[0], (((1,), (1,)), ((), ())),
                                   preferred_element_type=jnp.float32)


def _gmm(be, xsorted, w1t, w2t):
    return pl.pallas_call(
        _gmm_body,
        grid_spec=pltpu.PrefetchScalarGridSpec(
            num_scalar_prefetch=1,
            grid=(NBLK,),
            in_specs=[
                pl.BlockSpec((BLK, IDIM), lambda b, be_ref: (b, 0)),
                pl.BlockSpec((1, HID, IDIM), lambda b, be_ref: (be_ref[b], 0, 0)),
                pl.BlockSpec((1, IDIM, HID), lambda b, be_ref: (be_ref[b], 0, 0)),
            ],
            out_specs=pl.BlockSpec((BLK, IDIM), lambda b, be_ref: (b, 0)),
        ),
        out_shape=jax.ShapeDtypeStruct((NROWS, IDIM), jnp.float32),
        compiler_params=pltpu.CompilerParams(
            dimension_semantics=("arbitrary",),
            vmem_limit_bytes=100 * 1024 * 1024,
        ),
    )(be, xsorted, w1t, w2t)


def _combine_body(ys_hbm, d0_hbm, d1_hbm, w0_hbm, w1_hbm, out_hbm,
                  i0all, i1all, w0all, w1all, y0A, y1A, y0B, y1B, ov, semA, semB):
    wid = lax.axis_index("s") * 2 + lax.axis_index("c")
    base = wid * CHUNK
    lane = lax.iota(jnp.int32, 16)
    pltpu.sync_copy(d0_hbm.at[pl.ds(base, CHUNK)], i0all)
    pltpu.sync_copy(d1_hbm.at[pl.ds(base, CHUNK)], i1all)
    pltpu.sync_copy(w0_hbm.at[pl.ds(base, CHUNK)], w0all)
    pltpu.sync_copy(w1_hbm.at[pl.ds(base, CHUNK)], w1all)

    def start(y0, y1, sem, s):
        iv0 = i0all[pl.ds(s * SUB, SUB)]
        iv1 = i1all[pl.ds(s * SUB, SUB)]
        pltpu.async_copy(ys_hbm.at[iv0], y0, sem)
        pltpu.async_copy(ys_hbm.at[iv1], y1, sem)

    def finish(y0, y1, sem, s):
        dummy = i0all[pl.ds(0, SUB)]
        pltpu.make_async_copy(ys_hbm.at[dummy], y0, sem).wait()
        pltpu.make_async_copy(ys_hbm.at[dummy], y1, sem).wait()
        wa = w0all[pl.ds(s * SUB, SUB)]
        wb = w1all[pl.ds(s * SUB, SUB)]

        def tok(t, _):
            a = jnp.broadcast_to(jnp.sum(jnp.where(lane == t, wa, 0.0)), (16,))
            b = jnp.broadcast_to(jnp.sum(jnp.where(lane == t, wb, 0.0)), (16,))

            def colgrp(g, _):
                for u in range(8):
                    sl = pl.ds(g * 128 + u * 16, 16)
                    ov[t, sl] = a * y0[t, sl] + b * y1[t, sl]
                return 0

            lax.fori_loop(0, IDIM // 128, colgrp, 0)
            return 0

        lax.fori_loop(0, SUB, tok, 0)
        pltpu.sync_copy(ov, out_hbm.at[pl.ds(base + s * SUB, SUB)])

    start(y0A, y1A, semA, 0)
    start(y0B, y1B, semB, 1)

    def step(j2, _):
        sA = 2 * j2
        finish(y0A, y1A, semA, sA)

        @pl.when(j2 < NSTEP // 2 - 1)
        def _():
            start(y0A, y1A, semA, sA + 2)

        finish(y0B, y1B, semB, sA + 1)

        @pl.when(j2 < NSTEP // 2 - 1)
        def _():
            start(y0B, y1B, semB, sA + 3)

        return 0

    lax.fori_loop(0, NSTEP // 2, step, 0)


def _combine(ysorted, d0, d1, wt0, wt1):
    f = pl.kernel(
        _combine_body,
        out_type=jax.ShapeDtypeStruct((T, IDIM), jnp.float32),
        mesh=_sc_mesh(),
        scratch_types=[
            pltpu.VMEM((CHUNK,), jnp.int32),
            pltpu.VMEM((CHUNK,), jnp.int32),
            pltpu.VMEM((CHUNK,), jnp.float32),
            pltpu.VMEM((CHUNK,), jnp.float32),
            pltpu.VMEM((SUB, IDIM), jnp.float32),
            pltpu.VMEM((SUB, IDIM), jnp.float32),
            pltpu.VMEM((SUB, IDIM), jnp.float32),
            pltpu.VMEM((SUB, IDIM), jnp.float32),
            pltpu.VMEM((SUB, IDIM), jnp.float32),
            pltpu.SemaphoreType.DMA,
            pltpu.SemaphoreType.DMA,
        ],
        compiler_params=pltpu.CompilerParams(needs_layout_passes=False),
    )
    return f(ysorted, d0, d1, wt0, wt1)


def kernel(xs, gate_w, w1, w2):
    x = xs.reshape(-1, IDIM)
    d0, d1, wt0, wt1, ps2 = _route_meta(x, gate_w)
    ps = ps2[:, 0]
    starts = jnp.arange(NBLK, dtype=jnp.int32) * BLK
    be = jnp.sum((starts[:, None] >= ps[None, :]).astype(jnp.int32), axis=1) - 1
    xsorted = _dispatch(x, d0, d1)
    ysorted = _gmm(be, xsorted, w1.astype(jnp.bfloat16), w2.astype(jnp.bfloat16))
    out = _combine(ysorted, d0, d1, wt0, wt1)
    return out.reshape(xs.shape)
